# Initial kernel scaffold; baseline (speedup 1.0000x reference)
import jax
import jax.numpy as jnp
from jax.experimental import pallas as pl


def kernel(edge_index, topic_entity_one_hot, h_e, h_r, W1, b1, W2, b2):
    raise NotImplementedError("write your pallas kernel here")


# speedup vs baseline: 4.5861x; 4.5861x over previous
import jax
import jax.numpy as jnp
from jax import lax
from jax.experimental import pallas as pl
from jax.experimental.pallas import tpu as pltpu
from jax.experimental.pallas import tpu_sc as plsc

_NC = 2
_NS = 16
_NW = _NC * _NS


def _make_cnt_kernel(n, d, e, c):
    ep = e // _NW
    nch = ep // c
    mesh = plsc.VectorSubcoreMesh(core_axis_name="c", subcore_axis_name="s")

    def body(dst_hbm, zrows_hbm, ones_hbm, out_cnt, didx_v, ones_v, acc):
        ci = lax.axis_index("c")
        si = lax.axis_index("s")
        wid = ci * _NS + si

        @pl.when(si == 0)
        def _init():
            pltpu.sync_copy(zrows_hbm, acc)

        pltpu.sync_copy(ones_hbm, ones_v)
        plsc.subcore_barrier()
        base = wid * ep

        def step(j, carry):
            off = base + j * c
            pltpu.sync_copy(dst_hbm.at[pl.ds(off, c)], didx_v)
            pltpu.sync_copy(ones_v, acc.at[didx_v], add=True)
            return carry

        lax.fori_loop(0, nch, step, 0)
        plsc.subcore_barrier()

        @pl.when(si == 0)
        def _drain():
            pltpu.sync_copy(acc, out_cnt.at[pl.ds(ci * n, n)])

    return pl.kernel(
        body,
        mesh=mesh,
        out_type=jax.ShapeDtypeStruct((_NC * n, d), jnp.float32),
        scratch_types=[
            pltpu.VMEM((c,), jnp.int32),
            pltpu.VMEM((c, d), jnp.float32),
            pltpu.VMEM_SHARED((n, d), jnp.float32),
        ],
    )


def _make_hr_kernel(n, d, e, c):
    ep = e // _NW
    nch = ep // c
    mesh = plsc.VectorSubcoreMesh(core_axis_name="c", subcore_axis_name="s")

    def body(dst_hbm, hr_hbm, zrows_hbm, out_hbm, didx_v, rows_v, acc):
        ci = lax.axis_index("c")
        si = lax.axis_index("s")
        wid = ci * _NS + si

        @pl.when(si == 0)
        def _init():
            pltpu.sync_copy(zrows_hbm, acc)

        plsc.subcore_barrier()
        base = wid * ep

        def step(j, carry):
            off = base + j * c
            pltpu.sync_copy(dst_hbm.at[pl.ds(off, c)], didx_v)
            pltpu.sync_copy(hr_hbm.at[pl.ds(off, c)], rows_v)
            pltpu.sync_copy(rows_v, acc.at[didx_v], add=True)
            return carry

        lax.fori_loop(0, nch, step, 0)
        plsc.subcore_barrier()

        @pl.when(si == 0)
        def _drain():
            pltpu.sync_copy(acc, out_hbm.at[pl.ds(ci * n, n)])

    return pl.kernel(
        body,
        mesh=mesh,
        out_type=jax.ShapeDtypeStruct((_NC * n, d), jnp.float32),
        scratch_types=[
            pltpu.VMEM((c,), jnp.int32),
            pltpu.VMEM((c, d), jnp.float32),
            pltpu.VMEM_SHARED((n, d), jnp.float32),
        ],
    )


def _make_gather_kernel(n, d, e, c):
    ep = e // _NW
    nch = ep // c
    mesh = plsc.VectorSubcoreMesh(core_axis_name="c", subcore_axis_name="s")

    def body(src_hbm, dst_hbm, h_hbm, zrows_hbm, out_hbm,
             sidx_v, didx_v, rows_v, acc, sem):
        ci = lax.axis_index("c")
        si = lax.axis_index("s")
        wid = ci * _NS + si

        @pl.when(si == 0)
        def _init():
            pltpu.sync_copy(zrows_hbm, acc)

        plsc.subcore_barrier()
        base = wid * ep

        def step(j, carry):
            off = base + j * c
            pltpu.sync_copy(src_hbm.at[pl.ds(off, c)], sidx_v)
            pltpu.sync_copy(dst_hbm.at[pl.ds(off, c)], didx_v)
            pltpu.async_copy(h_hbm.at[sidx_v], rows_v, sem).wait()
            pltpu.sync_copy(rows_v, acc.at[didx_v], add=True)
            return carry

        lax.fori_loop(0, nch, step, 0)
        plsc.subcore_barrier()

        @pl.when(si == 0)
        def _drain():
            pltpu.sync_copy(acc, out_hbm.at[pl.ds(ci * n, n)])

    return pl.kernel(
        body,
        mesh=mesh,
        out_type=jax.ShapeDtypeStruct((_NC * n, d), jnp.float32),
        scratch_types=[
            pltpu.VMEM((c,), jnp.int32),
            pltpu.VMEM((c,), jnp.int32),
            pltpu.VMEM((c, d), jnp.float32),
            pltpu.VMEM_SHARED((n, d), jnp.float32),
            pltpu.SemaphoreType.DMA,
        ],
    )


def _mlp_body(h_ref, sa_ref, sb_ref, ha_ref, hb_ref, ca_ref, cb_ref,
              w_ref, b_ref, o_ref):
    d = h_ref.shape[1]
    cnt = ca_ref[:, 0:1] + cb_ref[:, 0:1]
    inv = 1.0 / jnp.maximum(cnt, 1.0)
    he = (sa_ref[...] + sb_ref[...]) * inv
    hr = (ha_ref[...] + hb_ref[...]) * inv
    w = w_ref[...]
    acc = jnp.dot(h_ref[...], w[:d],
                  preferred_element_type=jnp.float32,
                  precision=lax.Precision.HIGHEST)
    acc = acc + jnp.dot(he, w[d:2 * d],
                        preferred_element_type=jnp.float32,
                        precision=lax.Precision.HIGHEST)
    acc = acc + jnp.dot(hr, w[2 * d:],
                        preferred_element_type=jnp.float32,
                        precision=lax.Precision.HIGHEST)
    o_ref[...] = jnp.maximum(acc + b_ref[...], 0.0)


def _mlp(h, seg2, hr2, cnt2, w, b):
    n, d = h.shape
    bn = 1000
    grid = (n // bn,)

    def row_spec(width):
        return pl.BlockSpec((bn, width), lambda i: (i, 0))

    const_w = pl.BlockSpec((3 * d, d), lambda i: (0, 0))
    const_b = pl.BlockSpec((1, d), lambda i: (0, 0))
    return pl.pallas_call(
        _mlp_body,
        grid=grid,
        in_specs=[row_spec(d), row_spec(d), row_spec(d), row_spec(d),
                  row_spec(d), row_spec(d), row_spec(d), const_w, const_b],
        out_specs=row_spec(d),
        out_shape=jax.ShapeDtypeStruct((n, d), jnp.float32),
    )(h, seg2[:n], seg2[n:], hr2[:n], hr2[n:], cnt2[:n], cnt2[n:],
      w, b.reshape(1, d))


def kernel(edge_index, topic_entity_one_hot, h_e, h_r, W1, b1, W2, b2):
    n, d = h_e.shape
    e = h_r.shape[0]
    c = 80
    src = edge_index[0]
    dst = edge_index[1]

    zrows = jnp.zeros((n, d), jnp.float32)
    ones = jnp.ones((c, d), jnp.float32)

    cnt2 = _make_cnt_kernel(n, d, e, c)(dst, zrows, ones)
    hr2 = _make_hr_kernel(n, d, e, c)(dst, h_r, zrows)
    gather = _make_gather_kernel(n, d, e, c)
    seg1 = gather(src, dst, h_e, zrows)
    h1 = _mlp(h_e, seg1, hr2, cnt2, W1, b1)
    seg2 = gather(src, dst, h1, zrows)
    h2 = _mlp(h1, seg2, hr2, cnt2, W2, b2)
    return (topic_entity_one_hot, h2)


# speedup vs baseline: 9.3749x; 2.0442x over previous
import jax
import jax.numpy as jnp
from jax import lax
from jax.experimental import pallas as pl
from jax.experimental.pallas import tpu as pltpu
from jax.experimental.pallas import tpu_sc as plsc

_NC = 2
_NS = 16
_NW = _NC * _NS


def _make_scatter_kernel(n, d, e, c, mode):
    ep = e // _NW
    nch = ep // c
    assert nch % 2 == 1 and nch >= 3
    mesh = plsc.VectorSubcoreMesh(core_axis_name="c", subcore_axis_name="s")

    def body(*refs):
        if mode == "cnt":
            (dst_hbm, zrows_hbm, ones_hbm, out_hbm,
             didx0, didx1, ones_v, acc, sd0, sd1) = refs
            didx = (didx0, didx1)
        elif mode == "linear":
            (dst_hbm, rows_hbm, zrows_hbm, out_hbm,
             didx0, didx1, rows0, rows1, acc, sd0, sd1, sg0, sg1) = refs
            didx = (didx0, didx1)
            rows = (rows0, rows1)
            sg = (sg0, sg1)
        else:
            (src_hbm, dst_hbm, rows_hbm, zrows_hbm, out_hbm,
             sidx_all, didx0, didx1, rows0, rows1, acc,
             sd0, sd1, sg0, sg1) = refs
            didx = (didx0, didx1)
            rows = (rows0, rows1)
            sg = (sg0, sg1)
        sd = (sd0, sd1)

        ci = lax.axis_index("c")
        si = lax.axis_index("s")
        wid = ci * _NS + si
        base = wid * ep

        if mode == "gather":
            pltpu.sync_copy(src_hbm.at[pl.ds(base, ep)], sidx_all)
        if mode == "cnt":
            pltpu.sync_copy(ones_hbm, ones_v)

        @pl.when(si == 0)
        def _init():
            pltpu.sync_copy(zrows_hbm, acc)

        plsc.subcore_barrier()

        def didx_copy(j, p):
            return pltpu.make_async_copy(
                dst_hbm.at[pl.ds(base + j * c, c)], didx[p], sd[p])

        def rows_copy(j, p):
            if mode == "linear":
                return pltpu.make_async_copy(
                    rows_hbm.at[pl.ds(base + j * c, c)], rows[p], sg[p])
            return pltpu.make_async_copy(
                rows_hbm.at[sidx_all.at[pl.ds(j * c, c)]], rows[p], sg[p])

        def issue(j, p):
            didx_copy(j, p).start()
            if mode != "cnt":
                rows_copy(j, p).start()

        def finish(j, p):
            didx_copy(j, p).wait()
            if mode != "cnt":
                rows_copy(j, p).wait()
                pltpu.sync_copy(rows[p], acc.at[didx[p]], add=True)
            else:
                pltpu.sync_copy(ones_v, acc.at[didx[p]], add=True)

        issue(0, 0)

        def step(jj, carry):
            j = 2 * jj
            issue(j + 1, 1)
            finish(j, 0)
            issue(j + 2, 0)
            finish(j + 1, 1)
            return carry

        lax.fori_loop(0, (nch - 1) // 2, step, 0)
        finish(nch - 1, 0)

        plsc.subcore_barrier()

        @pl.when(si == 0)
        def _drain():
            pltpu.sync_copy(acc, out_hbm.at[pl.ds(ci * n, n)])

    if mode == "cnt":
        scratch = [
            pltpu.VMEM((c,), jnp.int32),
            pltpu.VMEM((c,), jnp.int32),
            pltpu.VMEM((c, d), jnp.float32),
            pltpu.VMEM_SHARED((n, d), jnp.float32),
            pltpu.SemaphoreType.DMA,
            pltpu.SemaphoreType.DMA,
        ]
    else:
        scratch = ([pltpu.VMEM((ep,), jnp.int32)] if mode == "gather" else []) + [
            pltpu.VMEM((c,), jnp.int32),
            pltpu.VMEM((c,), jnp.int32),
            pltpu.VMEM((c, d), jnp.float32),
            pltpu.VMEM((c, d), jnp.float32),
            pltpu.VMEM_SHARED((n, d), jnp.float32),
            pltpu.SemaphoreType.DMA,
            pltpu.SemaphoreType.DMA,
            pltpu.SemaphoreType.DMA,
            pltpu.SemaphoreType.DMA,
        ]

    return pl.kernel(
        body,
        mesh=mesh,
        out_type=jax.ShapeDtypeStruct((_NC * n, d), jnp.float32),
        scratch_types=scratch,
    )


def _mlp_body(h_ref, sa_ref, sb_ref, ha_ref, hb_ref, ca_ref, cb_ref,
              w_ref, b_ref, o_ref):
    d = h_ref.shape[1]
    cnt = ca_ref[:, 0:1] + cb_ref[:, 0:1]
    inv = 1.0 / jnp.maximum(cnt, 1.0)
    he = (sa_ref[...] + sb_ref[...]) * inv
    hr = (ha_ref[...] + hb_ref[...]) * inv
    w = w_ref[...]
    acc = jnp.dot(h_ref[...], w[:d],
                  preferred_element_type=jnp.float32,
                  precision=lax.Precision.HIGHEST)
    acc = acc + jnp.dot(he, w[d:2 * d],
                        preferred_element_type=jnp.float32,
                        precision=lax.Precision.HIGHEST)
    acc = acc + jnp.dot(hr, w[2 * d:],
                        preferred_element_type=jnp.float32,
                        precision=lax.Precision.HIGHEST)
    o_ref[...] = jnp.maximum(acc + b_ref[...], 0.0)


def _mlp(h, seg2, hr2, cnt2, w, b):
    n, d = h.shape
    bn = 1000
    grid = (n // bn,)

    def row_spec(width):
        return pl.BlockSpec((bn, width), lambda i: (i, 0))

    const_w = pl.BlockSpec((3 * d, d), lambda i: (0, 0))
    const_b = pl.BlockSpec((1, d), lambda i: (0, 0))
    return pl.pallas_call(
        _mlp_body,
        grid=grid,
        in_specs=[row_spec(d), row_spec(d), row_spec(d), row_spec(d),
                  row_spec(d), row_spec(d), row_spec(d), const_w, const_b],
        out_specs=row_spec(d),
        out_shape=jax.ShapeDtypeStruct((n, d), jnp.float32),
    )(h, seg2[:n], seg2[n:], hr2[:n], hr2[n:], cnt2[:n], cnt2[n:],
      w, b.reshape(1, d))


def kernel(edge_index, topic_entity_one_hot, h_e, h_r, W1, b1, W2, b2):
    n, d = h_e.shape
    e = h_r.shape[0]
    c = 80
    src = edge_index[0]
    dst = edge_index[1]

    zrows = jnp.zeros((n, d), jnp.float32)
    ones = jnp.ones((c, d), jnp.float32)

    cnt2 = _make_scatter_kernel(n, d, e, c, "cnt")(dst, zrows, ones)
    hr2 = _make_scatter_kernel(n, d, e, c, "linear")(dst, h_r, zrows)
    gather = _make_scatter_kernel(n, d, e, c, "gather")
    seg1 = gather(src, dst, h_e, zrows)
    h1 = _mlp(h_e, seg1, hr2, cnt2, W1, b1)
    seg2 = gather(src, dst, h1, zrows)
    h2 = _mlp(h1, seg2, hr2, cnt2, W2, b2)
    return (topic_entity_one_hot, h2)


# speedup vs baseline: 10.4092x; 1.1103x over previous
import jax
import jax.numpy as jnp
from jax import lax
from jax.experimental import pallas as pl
from jax.experimental.pallas import tpu as pltpu
from jax.experimental.pallas import tpu_sc as plsc

_NC = 2
_NS = 16
_NW = _NC * _NS


def _make_scatter_kernel(n, d, e, c, mode):
    ep = e // _NW
    nch = ep // c
    nb = 4
    assert (nch - 1) % nb == 0 and nch > 2 * nb
    mesh = plsc.VectorSubcoreMesh(core_axis_name="c", subcore_axis_name="s")

    def body(*refs):
        it = iter(refs)

        def take(k):
            return tuple(next(it) for _ in range(k))

        if mode == "cnt":
            dst_hbm, zrows_hbm, ones_hbm, out_hbm = take(4)
            didx = take(nb)
            (ones_v, acc) = take(2)
            sd = take(nb)
            ss = take(nb)
        elif mode == "linear":
            dst_hbm, rows_hbm, zrows_hbm, out_hbm = take(4)
            didx = take(nb)
            rows = take(nb)
            acc, = take(1)
            sd = take(nb)
            sg = take(nb)
            ss = take(nb)
        else:
            src_hbm, dst_hbm, rows_hbm, zrows_hbm, out_hbm = take(5)
            sidx = take(nb)
            didx = take(nb)
            rows = take(nb)
            acc, = take(1)
            si_ = take(nb)
            sd = take(nb)
            sg = take(nb)
            ss = take(nb)

        ci = lax.axis_index("c")
        s_ax = lax.axis_index("s")
        wid = ci * _NS + s_ax
        base = wid * ep

        if mode == "cnt":
            pltpu.sync_copy(ones_hbm, ones_v)

        @pl.when(s_ax == 0)
        def _init():
            pltpu.sync_copy(zrows_hbm, acc)

        plsc.subcore_barrier()

        def didx_copy(j, p):
            return pltpu.make_async_copy(
                dst_hbm.at[pl.ds(base + j * c, c)], didx[p], sd[p])

        def sidx_copy(j, p):
            return pltpu.make_async_copy(
                src_hbm.at[pl.ds(base + j * c, c)], sidx[p], si_[p])

        def lrows_copy(j, p):
            return pltpu.make_async_copy(
                rows_hbm.at[pl.ds(base + j * c, c)], rows[p], sg[p])

        def grows_copy(j, p):
            return pltpu.make_async_copy(
                rows_hbm.at[sidx[p]], rows[p], sg[p])

        def scatter_start(j, p):
            src = ones_v if mode == "cnt" else rows[p]
            pltpu.async_copy(src, acc.at[didx[p]], ss[p], add=True)

        def scatter_wait(j, p):
            src = ones_v if mode == "cnt" else rows[p]
            pltpu.make_async_copy(src, acc.at[didx[p]], ss[p]).wait()

        def stage1(k, p, guard_issue, guard_swait):

            def _swait():
                scatter_wait(k - nb, p)

            def _start():
                didx_copy(k, p).start()
                if mode == "gather":
                    sidx_copy(k, p).start()
                elif mode == "linear":
                    lrows_copy(k, p).start()

            if guard_swait:
                pl.when(k >= nb)(_swait)
            else:
                _swait()
            if guard_issue:
                pl.when(k < nch)(_start)
            else:
                _start()

        def stage2(k, p):
            if mode != "gather":
                return
            sidx_copy(k, p).wait()
            grows_copy(k, p).start()

        def stage3(k, p):
            didx_copy(k, p).wait()
            if mode == "linear":
                lrows_copy(k, p).wait()
            elif mode == "gather":
                grows_copy(k, p).wait()
            scatter_start(k, p)

        stage1(0, 0, False, True)
        stage1(1, 1, False, True)
        stage2(0, 0)

        def step(jj, carry):
            kk = nb * jj
            for p in range(nb):
                k = kk + p
                stage1(k + 2, (p + 2) % nb,
                       guard_issue=(p == nb - 1), guard_swait=(p < 2))
                stage2(k + 1, (p + 1) % nb)
                stage3(k, p)
            return carry

        lax.fori_loop(0, (nch - 1) // nb, step, 0)
        stage3(nch - 1, (nch - 1) % nb)
        for back in range(3):
            j = nch - 1 - back
            scatter_wait(j, j % nb)

        plsc.subcore_barrier()

        @pl.when(s_ax == 0)
        def _drain():
            pltpu.sync_copy(acc, out_hbm.at[pl.ds(ci * n, n)])

    sems = [pltpu.SemaphoreType.DMA] * nb
    idxbuf = [pltpu.VMEM((c,), jnp.int32)] * nb
    rowbuf = [pltpu.VMEM((c, d), jnp.float32)] * nb
    shared = [pltpu.VMEM_SHARED((n, d), jnp.float32)]
    if mode == "cnt":
        scratch = (idxbuf + [pltpu.VMEM((c, d), jnp.float32)] + shared
                   + sems + sems)
    elif mode == "linear":
        scratch = idxbuf + rowbuf + shared + sems + sems + sems
    else:
        scratch = (idxbuf + idxbuf + rowbuf + shared
                   + sems + sems + sems + sems)

    return pl.kernel(
        body,
        mesh=mesh,
        out_type=jax.ShapeDtypeStruct((_NC * n, d), jnp.float32),
        scratch_types=scratch,
    )


def _mlp_body(h_ref, sa_ref, sb_ref, ha_ref, hb_ref, ca_ref, cb_ref,
              w_ref, b_ref, o_ref):
    d = h_ref.shape[1]
    cnt = ca_ref[:, 0:1] + cb_ref[:, 0:1]
    inv = 1.0 / jnp.maximum(cnt, 1.0)
    he = (sa_ref[...] + sb_ref[...]) * inv
    hr = (ha_ref[...] + hb_ref[...]) * inv
    w = w_ref[...]
    acc = jnp.dot(h_ref[...], w[:d],
                  preferred_element_type=jnp.float32,
                  precision=lax.Precision.HIGHEST)
    acc = acc + jnp.dot(he, w[d:2 * d],
                        preferred_element_type=jnp.float32,
                        precision=lax.Precision.HIGHEST)
    acc = acc + jnp.dot(hr, w[2 * d:],
                        preferred_element_type=jnp.float32,
                        precision=lax.Precision.HIGHEST)
    o_ref[...] = jnp.maximum(acc + b_ref[...], 0.0)


def _mlp(h, seg2, hr2, cnt2, w, b):
    n, d = h.shape
    bn = 1000
    grid = (n // bn,)

    def row_spec(width):
        return pl.BlockSpec((bn, width), lambda i: (i, 0))

    const_w = pl.BlockSpec((3 * d, d), lambda i: (0, 0))
    const_b = pl.BlockSpec((1, d), lambda i: (0, 0))
    return pl.pallas_call(
        _mlp_body,
        grid=grid,
        in_specs=[row_spec(d), row_spec(d), row_spec(d), row_spec(d),
                  row_spec(d), row_spec(d), row_spec(d), const_w, const_b],
        out_specs=row_spec(d),
        out_shape=jax.ShapeDtypeStruct((n, d), jnp.float32),
    )(h, seg2[:n], seg2[n:], hr2[:n], hr2[n:], cnt2[:n], cnt2[n:],
      w, b.reshape(1, d))


def kernel(edge_index, topic_entity_one_hot, h_e, h_r, W1, b1, W2, b2):
    n, d = h_e.shape
    e = h_r.shape[0]
    c = 80
    src = edge_index[0]
    dst = edge_index[1]

    zrows = jnp.zeros((n, d), jnp.float32)
    ones = jnp.ones((c, d), jnp.float32)

    cnt2 = _make_scatter_kernel(n, d, e, c, "cnt")(dst, zrows, ones)
    hr2 = _make_scatter_kernel(n, d, e, c, "linear")(dst, h_r, zrows)
    gather = _make_scatter_kernel(n, d, e, c, "gather")
    seg1 = gather(src, dst, h_e, zrows)
    h1 = _mlp(h_e, seg1, hr2, cnt2, W1, b1)
    seg2 = gather(src, dst, h1, zrows)
    h2 = _mlp(h1, seg2, hr2, cnt2, W2, b2)
    return (topic_entity_one_hot, h2)


# speedup vs baseline: 10.9067x; 1.0478x over previous
import jax
import jax.numpy as jnp
from jax import lax
from jax.experimental import pallas as pl
from jax.experimental.pallas import tpu as pltpu
from jax.experimental.pallas import tpu_sc as plsc

_NC = 2
_NS = 16
_NW = _NC * _NS


def _make_scatter_kernel(n, d, e, c, mode):
    ep = e // _NW
    nch = ep // c
    nb = 4
    assert (nch - 1) % nb == 0 and nch > 2 * nb
    mesh = plsc.VectorSubcoreMesh(core_axis_name="c", subcore_axis_name="s")

    def body(*refs):
        it = iter(refs)

        def take(k):
            return tuple(next(it) for _ in range(k))

        if mode == "cnt":
            dst_hbm, zrows_hbm, ones_hbm, out_a, out_b = take(5)
            didx = take(nb)
            (ones_v, acc) = take(2)
            sd = take(nb)
            ss = take(nb)
        elif mode == "linear":
            dst_hbm, rows_hbm, zrows_hbm, out_a, out_b = take(5)
            didx = take(nb)
            rows = take(nb)
            acc, = take(1)
            sd = take(nb)
            sg = take(nb)
            ss = take(nb)
        else:
            src_hbm, dst_hbm, rows_hbm, zrows_hbm, out_a, out_b = take(6)
            sidx = take(nb)
            didx = take(nb)
            rows = take(nb)
            acc, = take(1)
            si_ = take(nb)
            sd = take(nb)
            sg = take(nb)
            ss = take(nb)

        ci = lax.axis_index("c")
        s_ax = lax.axis_index("s")
        wid = ci * _NS + s_ax
        base = wid * ep

        if mode == "cnt":
            pltpu.sync_copy(ones_hbm, ones_v)

        @pl.when(s_ax == 0)
        def _init():
            pltpu.sync_copy(zrows_hbm, acc)

        plsc.subcore_barrier()

        def didx_copy(j, p):
            return pltpu.make_async_copy(
                dst_hbm.at[pl.ds(base + j * c, c)], didx[p], sd[p])

        def sidx_copy(j, p):
            return pltpu.make_async_copy(
                src_hbm.at[pl.ds(base + j * c, c)], sidx[p], si_[p])

        def lrows_copy(j, p):
            return pltpu.make_async_copy(
                rows_hbm.at[pl.ds(base + j * c, c)], rows[p], sg[p])

        def grows_copy(j, p):
            return pltpu.make_async_copy(
                rows_hbm.at[sidx[p]], rows[p], sg[p])

        def scatter_start(j, p):
            src = ones_v if mode == "cnt" else rows[p]
            pltpu.async_copy(src, acc.at[didx[p]], ss[p], add=True)

        def scatter_wait(j, p):
            src = ones_v if mode == "cnt" else rows[p]
            pltpu.make_async_copy(src, acc.at[didx[p]], ss[p]).wait()

        def stage1(k, p, guard_issue, guard_swait):

            def _swait():
                scatter_wait(k - nb, p)

            def _start():
                didx_copy(k, p).start()
                if mode == "gather":
                    sidx_copy(k, p).start()
                elif mode == "linear":
                    lrows_copy(k, p).start()

            if guard_swait:
                pl.when(k >= nb)(_swait)
            else:
                _swait()
            if guard_issue:
                pl.when(k < nch)(_start)
            else:
                _start()

        def stage2(k, p):
            if mode != "gather":
                return
            sidx_copy(k, p).wait()
            grows_copy(k, p).start()

        def stage3(k, p):
            didx_copy(k, p).wait()
            if mode == "linear":
                lrows_copy(k, p).wait()
            elif mode == "gather":
                grows_copy(k, p).wait()
            scatter_start(k, p)

        stage1(0, 0, False, True)
        stage1(1, 1, False, True)
        stage2(0, 0)

        def step(jj, carry):
            kk = nb * jj
            for p in range(nb):
                k = kk + p
                stage1(k + 2, (p + 2) % nb,
                       guard_issue=(p == nb - 1), guard_swait=(p < 2))
                stage2(k + 1, (p + 1) % nb)
                stage3(k, p)
            return carry

        lax.fori_loop(0, (nch - 1) // nb, step, 0)
        stage3(nch - 1, (nch - 1) % nb)
        for back in range(3):
            j = nch - 1 - back
            scatter_wait(j, j % nb)

        plsc.subcore_barrier()

        @pl.when((s_ax == 0) & (ci == 0))
        def _drain_a():
            pltpu.sync_copy(acc, out_a)

        @pl.when((s_ax == 0) & (ci == 1))
        def _drain_b():
            pltpu.sync_copy(acc, out_b)

    sems = [pltpu.SemaphoreType.DMA] * nb
    idxbuf = [pltpu.VMEM((c,), jnp.int32)] * nb
    rowbuf = [pltpu.VMEM((c, d), jnp.float32)] * nb
    shared = [pltpu.VMEM_SHARED((n, d), jnp.float32)]
    if mode == "cnt":
        scratch = (idxbuf + [pltpu.VMEM((c, d), jnp.float32)] + shared
                   + sems + sems)
    elif mode == "linear":
        scratch = idxbuf + rowbuf + shared + sems + sems + sems
    else:
        scratch = (idxbuf + idxbuf + rowbuf + shared
                   + sems + sems + sems + sems)

    return pl.kernel(
        body,
        mesh=mesh,
        out_type=(jax.ShapeDtypeStruct((n, d), jnp.float32),
                  jax.ShapeDtypeStruct((n, d), jnp.float32)),
        scratch_types=scratch,
    )


def _mlp_body(h_ref, sa_ref, sb_ref, ha_ref, hb_ref, ca_ref, cb_ref,
              w_ref, b_ref, o_ref):
    d = h_ref.shape[1]
    cnt = ca_ref[:, 0:1] + cb_ref[:, 0:1]
    inv = 1.0 / jnp.maximum(cnt, 1.0)
    he = (sa_ref[...] + sb_ref[...]) * inv
    hr = (ha_ref[...] + hb_ref[...]) * inv
    w = w_ref[...]
    acc = jnp.dot(h_ref[...], w[:d],
                  preferred_element_type=jnp.float32,
                  precision=lax.Precision.HIGHEST)
    acc = acc + jnp.dot(he, w[d:2 * d],
                        preferred_element_type=jnp.float32,
                        precision=lax.Precision.HIGHEST)
    acc = acc + jnp.dot(hr, w[2 * d:],
                        preferred_element_type=jnp.float32,
                        precision=lax.Precision.HIGHEST)
    o_ref[...] = jnp.maximum(acc + b_ref[...], 0.0)


def _mlp(h, sega, segb, hra, hrb, cnta, cntb, w, b):
    n, d = h.shape
    bn = 1000
    grid = (n // bn,)

    def row_spec(width):
        return pl.BlockSpec((bn, width), lambda i: (i, 0))

    const_w = pl.BlockSpec((3 * d, d), lambda i: (0, 0))
    const_b = pl.BlockSpec((1, d), lambda i: (0, 0))
    return pl.pallas_call(
        _mlp_body,
        grid=grid,
        in_specs=[row_spec(d), row_spec(d), row_spec(d), row_spec(d),
                  row_spec(d), row_spec(d), row_spec(d), const_w, const_b],
        out_specs=row_spec(d),
        out_shape=jax.ShapeDtypeStruct((n, d), jnp.float32),
    )(h, sega, segb, hra, hrb, cnta, cntb,
      w, b.reshape(1, d))


def kernel(edge_index, topic_entity_one_hot, h_e, h_r, W1, b1, W2, b2):
    n, d = h_e.shape
    e = h_r.shape[0]
    c = 80
    src = edge_index[0]
    dst = edge_index[1]

    zrows = jnp.zeros((n, d), jnp.float32)

    ones = jnp.ones((c, d), jnp.float32)
    cnta, cntb = _make_scatter_kernel(n, d, e, c, "cnt")(dst, zrows, ones)
    hra, hrb = _make_scatter_kernel(n, d, e, c, "linear")(dst, h_r, zrows)
    gather = _make_scatter_kernel(n, d, e, c, "gather")
    s1a, s1b = gather(src, dst, h_e, zrows)
    h1 = _mlp(h_e, s1a, s1b, hra, hrb, cnta, cntb, W1, b1)
    s2a, s2b = gather(src, dst, h1, zrows)
    h2 = _mlp(h1, s2a, s2b, hra, hrb, cnta, cntb, W2, b2)
    return (topic_entity_one_hot, h2)


# speedup vs baseline: 11.0681x; 1.0148x over previous
import jax
import jax.numpy as jnp
from jax import lax
from jax.experimental import pallas as pl
from jax.experimental.pallas import tpu as pltpu
from jax.experimental.pallas import tpu_sc as plsc

_NC = 2
_NS = 16
_NW = _NC * _NS


def _make_scatter_kernel(n, d, e, c, mode):
    ep = e // _NW
    nch = ep // c
    nb = 4
    assert (nch - 1) % nb == 0 and nch > 2 * nb
    mesh = plsc.VectorSubcoreMesh(core_axis_name="c", subcore_axis_name="s")

    def body(*refs):
        it = iter(refs)

        def take(k):
            return tuple(next(it) for _ in range(k))

        if mode == "cnt":
            dst_hbm, zrows_hbm, ones_hbm, out_a, out_b = take(5)
            didx = take(nb)
            (ones_v, acc) = take(2)
            sd = take(nb)
            ss = take(nb)
        elif mode == "linear":
            dst_hbm, rows_hbm, zrows_hbm, out_a, out_b = take(5)
            didx = take(nb)
            rows = take(nb)
            acc, = take(1)
            sd = take(nb)
            sg = take(nb)
            ss = take(nb)
        else:
            src_hbm, dst_hbm, rows_hbm, zrows_hbm, out_a, out_b = take(6)
            sidx = take(nb)
            didx = take(nb)
            rows = take(nb)
            acc, = take(1)
            si_ = take(nb)
            sd = take(nb)
            sg = take(nb)
            ss = take(nb)

        ci = lax.axis_index("c")
        s_ax = lax.axis_index("s")
        wid = ci * _NS + s_ax
        base = wid * ep

        if mode == "cnt":
            pltpu.sync_copy(ones_hbm, ones_v)

        @pl.when(s_ax == 0)
        def _init():
            pltpu.sync_copy(zrows_hbm, acc)

        plsc.subcore_barrier()

        def didx_copy(j, p):
            return pltpu.make_async_copy(
                dst_hbm.at[pl.ds(base + j * c, c)], didx[p], sd[p])

        def sidx_copy(j, p):
            return pltpu.make_async_copy(
                src_hbm.at[pl.ds(base + j * c, c)], sidx[p], si_[p])

        def lrows_copy(j, p):
            return pltpu.make_async_copy(
                rows_hbm.at[pl.ds(base + j * c, c)], rows[p], sg[p])

        def grows_copy(j, p):
            return pltpu.make_async_copy(
                rows_hbm.at[sidx[p]], rows[p], sg[p])

        def scatter_start(j, p):
            src = ones_v if mode == "cnt" else rows[p]
            pltpu.async_copy(src, acc.at[didx[p]], ss[p], add=True)

        def scatter_wait(j, p):
            src = ones_v if mode == "cnt" else rows[p]
            pltpu.make_async_copy(src, acc.at[didx[p]], ss[p]).wait()

        def stage1(k, p, guard_issue, guard_swait):

            def _swait():
                scatter_wait(k - nb, p)

            def _start():
                didx_copy(k, p).start()
                if mode == "gather":
                    sidx_copy(k, p).start()
                elif mode == "linear":
                    lrows_copy(k, p).start()

            if guard_swait:
                pl.when(k >= nb)(_swait)
            else:
                _swait()
            if guard_issue:
                pl.when(k < nch)(_start)
            else:
                _start()

        def stage2(k, p):
            if mode != "gather":
                return
            sidx_copy(k, p).wait()
            grows_copy(k, p).start()

        def stage3(k, p):
            didx_copy(k, p).wait()
            if mode == "linear":
                lrows_copy(k, p).wait()
            elif mode == "gather":
                grows_copy(k, p).wait()
            scatter_start(k, p)

        stage1(0, 0, False, True)
        stage1(1, 1, False, True)
        stage2(0, 0)

        def step(jj, carry):
            kk = nb * jj
            for p in range(nb):
                k = kk + p
                stage1(k + 2, (p + 2) % nb,
                       guard_issue=(p == nb - 1), guard_swait=(p < 2))
                stage2(k + 1, (p + 1) % nb)
                stage3(k, p)
            return carry

        lax.fori_loop(0, (nch - 1) // nb, step, 0)
        stage3(nch - 1, (nch - 1) % nb)
        for back in range(3):
            j = nch - 1 - back
            scatter_wait(j, j % nb)

        plsc.subcore_barrier()

        @pl.when((s_ax == 0) & (ci == 0))
        def _drain_a():
            pltpu.sync_copy(acc, out_a)

        @pl.when((s_ax == 0) & (ci == 1))
        def _drain_b():
            pltpu.sync_copy(acc, out_b)

    sems = [pltpu.SemaphoreType.DMA] * nb
    idxbuf = [pltpu.VMEM((c,), jnp.int32)] * nb
    rowbuf = [pltpu.VMEM((c, d), jnp.float32)] * nb
    shared = [pltpu.VMEM_SHARED((n, d), jnp.float32)]
    if mode == "cnt":
        scratch = (idxbuf + [pltpu.VMEM((c, d), jnp.float32)] + shared
                   + sems + sems)
    elif mode == "linear":
        scratch = idxbuf + rowbuf + shared + sems + sems + sems
    else:
        scratch = (idxbuf + idxbuf + rowbuf + shared
                   + sems + sems + sems + sems)

    return pl.kernel(
        body,
        mesh=mesh,
        out_type=(jax.ShapeDtypeStruct((n, d), jnp.float32),
                  jax.ShapeDtypeStruct((n, d), jnp.float32)),
        scratch_types=scratch,
    )


def _make_stage1_kernel(n, d, e, c):
    ep = e // _NW
    nch = ep // c
    nb = 4
    assert (nch - 1) % nb == 0 and nch > 2 * nb
    mesh = plsc.VectorSubcoreMesh(core_axis_name="c", subcore_axis_name="s")

    def body(src_hbm, dst_hbm, hr_hbm, he_hbm, zrows_hbm, ones_hbm,
             cnt_a, cnt_b, hr_a, hr_b, seg_a, seg_b, *rest):
        sidx = rest[0:nb]
        didx = rest[nb:2 * nb]
        rows = rest[2 * nb:3 * nb]
        acc = rest[3 * nb]
        si_ = rest[3 * nb + 1:4 * nb + 1]
        sd = rest[4 * nb + 1:5 * nb + 1]
        sg = rest[5 * nb + 1:6 * nb + 1]
        ss = rest[6 * nb + 1:7 * nb + 1]

        ci = lax.axis_index("c")
        s_ax = lax.axis_index("s")
        wid = ci * _NS + s_ax
        base = wid * ep

        def init_acc():
            @pl.when(s_ax == 0)
            def _():
                pltpu.sync_copy(zrows_hbm, acc)

        def drain(out_a, out_b):
            @pl.when((s_ax == 0) & (ci == 0))
            def _a():
                pltpu.sync_copy(acc, out_a)

            @pl.when((s_ax == 0) & (ci == 1))
            def _b():
                pltpu.sync_copy(acc, out_b)

        def didx_copy(j, p):
            return pltpu.make_async_copy(
                dst_hbm.at[pl.ds(base + j * c, c)], didx[p], sd[p])

        def sidx_copy(j, p):
            return pltpu.make_async_copy(
                src_hbm.at[pl.ds(base + j * c, c)], sidx[p], si_[p])

        def run_pass(mode, rows_hbm):
            def lrows_copy(j, p):
                return pltpu.make_async_copy(
                    rows_hbm.at[pl.ds(base + j * c, c)], rows[p], sg[p])

            def grows_copy(j, p):
                return pltpu.make_async_copy(
                    rows_hbm.at[sidx[p]], rows[p], sg[p])

            def ssrc(p):
                return rows[0] if mode == "cnt" else rows[p]

            def scatter_start(j, p):
                pltpu.async_copy(ssrc(p), acc.at[didx[p]], ss[p], add=True)

            def scatter_wait(j, p):
                pltpu.make_async_copy(ssrc(p), acc.at[didx[p]], ss[p]).wait()

            def stage1(k, p, guard_issue, guard_swait):
                def _swait():
                    scatter_wait(k - nb, p)

                def _start():
                    didx_copy(k, p).start()
                    if mode == "gather":
                        sidx_copy(k, p).start()
                    elif mode == "linear":
                        lrows_copy(k, p).start()

                if guard_swait:
                    pl.when(k >= nb)(_swait)
                else:
                    _swait()
                if guard_issue:
                    pl.when(k < nch)(_start)
                else:
                    _start()

            def stage2(k, p):
                if mode != "gather":
                    return
                sidx_copy(k, p).wait()
                grows_copy(k, p).start()

            def stage3(k, p):
                didx_copy(k, p).wait()
                if mode == "linear":
                    lrows_copy(k, p).wait()
                elif mode == "gather":
                    grows_copy(k, p).wait()
                scatter_start(k, p)

            stage1(0, 0, False, True)
            stage1(1, 1, False, True)
            stage2(0, 0)

            def step(jj, carry):
                kk = nb * jj
                for p in range(nb):
                    k = kk + p
                    stage1(k + 2, (p + 2) % nb,
                           guard_issue=(p == nb - 1), guard_swait=(p < 2))
                    stage2(k + 1, (p + 1) % nb)
                    stage3(k, p)
                return carry

            lax.fori_loop(0, (nch - 1) // nb, step, 0)
            stage3(nch - 1, (nch - 1) % nb)
            for back in range(3):
                j = nch - 1 - back
                scatter_wait(j, j % nb)

        pltpu.sync_copy(ones_hbm, rows[0])
        init_acc()
        plsc.subcore_barrier()
        run_pass("cnt", None)
        plsc.subcore_barrier()
        drain(cnt_a, cnt_b)
        init_acc()
        plsc.subcore_barrier()
        run_pass("linear", hr_hbm)
        plsc.subcore_barrier()
        drain(hr_a, hr_b)
        init_acc()
        plsc.subcore_barrier()
        run_pass("gather", he_hbm)
        plsc.subcore_barrier()
        drain(seg_a, seg_b)

    shp = jax.ShapeDtypeStruct((n, d), jnp.float32)
    return pl.kernel(
        body,
        mesh=mesh,
        out_type=(shp,) * 6,
        scratch_types=([pltpu.VMEM((c,), jnp.int32)] * (2 * nb)
                       + [pltpu.VMEM((c, d), jnp.float32)] * nb
                       + [pltpu.VMEM_SHARED((n, d), jnp.float32)]
                       + [pltpu.SemaphoreType.DMA] * (4 * nb)),
    )


def _mlp_body(h_ref, sa_ref, sb_ref, ha_ref, hb_ref, ca_ref, cb_ref,
              w_ref, b_ref, o_ref):
    d = h_ref.shape[1]
    cnt = ca_ref[:, 0:1] + cb_ref[:, 0:1]
    inv = 1.0 / jnp.maximum(cnt, 1.0)
    he = (sa_ref[...] + sb_ref[...]) * inv
    hr = (ha_ref[...] + hb_ref[...]) * inv
    w = w_ref[...]
    acc = jnp.dot(h_ref[...], w[:d],
                  preferred_element_type=jnp.float32,
                  precision=lax.Precision.HIGHEST)
    acc = acc + jnp.dot(he, w[d:2 * d],
                        preferred_element_type=jnp.float32,
                        precision=lax.Precision.HIGHEST)
    acc = acc + jnp.dot(hr, w[2 * d:],
                        preferred_element_type=jnp.float32,
                        precision=lax.Precision.HIGHEST)
    o_ref[...] = jnp.maximum(acc + b_ref[...], 0.0)


def _mlp(h, sega, segb, hra, hrb, cnta, cntb, w, b):
    n, d = h.shape
    bn = 1000
    grid = (n // bn,)

    def row_spec(width):
        return pl.BlockSpec((bn, width), lambda i: (i, 0))

    const_w = pl.BlockSpec((3 * d, d), lambda i: (0, 0))
    const_b = pl.BlockSpec((1, d), lambda i: (0, 0))
    return pl.pallas_call(
        _mlp_body,
        grid=grid,
        in_specs=[row_spec(d), row_spec(d), row_spec(d), row_spec(d),
                  row_spec(d), row_spec(d), row_spec(d), const_w, const_b],
        out_specs=row_spec(d),
        out_shape=jax.ShapeDtypeStruct((n, d), jnp.float32),
    )(h, sega, segb, hra, hrb, cnta, cntb,
      w, b.reshape(1, d))


def kernel(edge_index, topic_entity_one_hot, h_e, h_r, W1, b1, W2, b2):
    n, d = h_e.shape
    e = h_r.shape[0]
    c = 80
    src = edge_index[0]
    dst = edge_index[1]

    zrows = jnp.zeros((n, d), jnp.float32)

    ones = jnp.ones((c, d), jnp.float32)
    cnta, cntb, hra, hrb, s1a, s1b = _make_stage1_kernel(n, d, e, c)(
        src, dst, h_r, h_e, zrows, ones)
    gather = _make_scatter_kernel(n, d, e, c, "gather")
    h1 = _mlp(h_e, s1a, s1b, hra, hrb, cnta, cntb, W1, b1)
    s2a, s2b = gather(src, dst, h1, zrows)
    h2 = _mlp(h1, s2a, s2b, hra, hrb, cnta, cntb, W2, b2)
    return (topic_entity_one_hot, h2)


# speedup vs baseline: 11.3837x; 1.0285x over previous
import jax
import jax.numpy as jnp
from jax import lax
from jax.experimental import pallas as pl
from jax.experimental.pallas import tpu as pltpu
from jax.experimental.pallas import tpu_sc as plsc

_NC = 2
_NS = 16
_NW = _NC * _NS


def _make_scatter_kernel(n, d, e, c, mode):
    ep = e // _NW
    nch = ep // c
    nb = 4
    assert (nch - 1) % nb == 0 and nch > 2 * nb
    mesh = plsc.VectorSubcoreMesh(core_axis_name="c", subcore_axis_name="s")

    def body(*refs):
        it = iter(refs)

        def take(k):
            return tuple(next(it) for _ in range(k))

        if mode == "cnt":
            dst_hbm, zrows_hbm, ones_hbm, out_a, out_b = take(5)
            didx = take(nb)
            (ones_v, acc) = take(2)
            sd = take(nb)
            ss = take(nb)
        elif mode == "linear":
            dst_hbm, rows_hbm, zrows_hbm, out_a, out_b = take(5)
            didx = take(nb)
            rows = take(nb)
            acc, = take(1)
            sd = take(nb)
            sg = take(nb)
            ss = take(nb)
        else:
            src_hbm, dst_hbm, rows_hbm, prev_a, prev_b, out_a, out_b = take(7)
            sidx = take(nb)
            didx = take(nb)
            rows = take(nb)
            acc, = take(1)
            si_ = take(nb)
            sd = take(nb)
            sg = take(nb)
            ss = take(nb)

        ci = lax.axis_index("c")
        s_ax = lax.axis_index("s")
        wid = ci * _NS + s_ax
        base = wid * ep

        if mode == "cnt":
            pltpu.sync_copy(ones_hbm, ones_v)

        sz = (n // _NS + 7) // 8 * 8
        tail = n - (_NS - 1) * sz

        def spread(fn):
            @pl.when(s_ax < _NS - 1)
            def _main():
                fn(s_ax * sz, sz)

            @pl.when(s_ax == _NS - 1)
            def _tail():
                fn((_NS - 1) * sz, tail)

        def _ini(st, ln):
            @pl.when(ci == 0)
            def _a():
                pltpu.sync_copy(prev_a.at[pl.ds(st, ln)], acc.at[pl.ds(st, ln)])

            @pl.when(ci == 1)
            def _b():
                pltpu.sync_copy(prev_b.at[pl.ds(st, ln)], acc.at[pl.ds(st, ln)])

        spread(_ini)
        plsc.subcore_barrier()

        def didx_copy(j, p):
            return pltpu.make_async_copy(
                dst_hbm.at[pl.ds(base + j * c, c)], didx[p], sd[p])

        def sidx_copy(j, p):
            return pltpu.make_async_copy(
                src_hbm.at[pl.ds(base + j * c, c)], sidx[p], si_[p])

        def lrows_copy(j, p):
            return pltpu.make_async_copy(
                rows_hbm.at[pl.ds(base + j * c, c)], rows[p], sg[p])

        def grows_copy(j, p):
            return pltpu.make_async_copy(
                rows_hbm.at[sidx[p]], rows[p], sg[p])

        def scatter_start(j, p):
            src = ones_v if mode == "cnt" else rows[p]
            pltpu.async_copy(src, acc.at[didx[p]], ss[p], add=True)

        def scatter_wait(j, p):
            src = ones_v if mode == "cnt" else rows[p]
            pltpu.make_async_copy(src, acc.at[didx[p]], ss[p]).wait()

        def stage1(k, p, guard_issue, guard_swait):

            def _swait():
                scatter_wait(k - nb, p)

            def _start():
                didx_copy(k, p).start()
                if mode == "gather":
                    sidx_copy(k, p).start()
                elif mode == "linear":
                    lrows_copy(k, p).start()

            if guard_swait:
                pl.when(k >= nb)(_swait)
            else:
                _swait()
            if guard_issue:
                pl.when(k < nch)(_start)
            else:
                _start()

        def stage2(k, p):
            if mode != "gather":
                return
            sidx_copy(k, p).wait()
            grows_copy(k, p).start()

        def stage3(k, p):
            didx_copy(k, p).wait()
            if mode == "linear":
                lrows_copy(k, p).wait()
            elif mode == "gather":
                grows_copy(k, p).wait()
            scatter_start(k, p)

        stage1(0, 0, False, True)
        stage1(1, 1, False, True)
        stage2(0, 0)

        def step(jj, carry):
            kk = nb * jj
            for p in range(nb):
                k = kk + p
                stage1(k + 2, (p + 2) % nb,
                       guard_issue=(p == nb - 1), guard_swait=(p < 2))
                stage2(k + 1, (p + 1) % nb)
                stage3(k, p)
            return carry

        lax.fori_loop(0, (nch - 1) // nb, step, 0)
        stage3(nch - 1, (nch - 1) % nb)
        for back in range(3):
            j = nch - 1 - back
            scatter_wait(j, j % nb)

        plsc.subcore_barrier()

        def _go(st, ln):
            @pl.when(ci == 0)
            def _a():
                pltpu.sync_copy(acc.at[pl.ds(st, ln)], out_a.at[pl.ds(st, ln)])

            @pl.when(ci == 1)
            def _b():
                pltpu.sync_copy(acc.at[pl.ds(st, ln)], out_b.at[pl.ds(st, ln)])

        spread(_go)

    sems = [pltpu.SemaphoreType.DMA] * nb
    idxbuf = [pltpu.VMEM((c,), jnp.int32)] * nb
    rowbuf = [pltpu.VMEM((c, d), jnp.float32)] * nb
    shared = [pltpu.VMEM_SHARED((n, d), jnp.float32)]
    if mode == "cnt":
        scratch = (idxbuf + [pltpu.VMEM((c, d), jnp.float32)] + shared
                   + sems + sems)
    elif mode == "linear":
        scratch = idxbuf + rowbuf + shared + sems + sems + sems
    else:
        scratch = (idxbuf + idxbuf + rowbuf + shared
                   + sems + sems + sems + sems)

    return pl.kernel(
        body,
        mesh=mesh,
        out_type=(jax.ShapeDtypeStruct((n, d), jnp.float32),
                  jax.ShapeDtypeStruct((n, d), jnp.float32)),
        scratch_types=scratch,
    )


def _make_stage1_kernel(n, d, e, c):
    ep = e // _NW
    nch = ep // c
    nb = 4
    assert (nch - 1) % nb == 0 and nch > 2 * nb
    mesh = plsc.VectorSubcoreMesh(core_axis_name="c", subcore_axis_name="s")

    def body(src_hbm, dst_hbm, hr_hbm, he_hbm, zrows_hbm, ones_hbm,
             cnt_a, cnt_b, hr_a, hr_b, seg_a, seg_b, *rest):
        sidx = rest[0:nb]
        didx = rest[nb:2 * nb]
        rows = rest[2 * nb:3 * nb]
        acc = rest[3 * nb]
        si_ = rest[3 * nb + 1:4 * nb + 1]
        sd = rest[4 * nb + 1:5 * nb + 1]
        sg = rest[5 * nb + 1:6 * nb + 1]
        ss = rest[6 * nb + 1:7 * nb + 1]

        ci = lax.axis_index("c")
        s_ax = lax.axis_index("s")
        wid = ci * _NS + s_ax
        base = wid * ep

        sz = (n // _NS + 7) // 8 * 8
        tail = n - (_NS - 1) * sz

        def spread(fn):
            @pl.when(s_ax < _NS - 1)
            def _main():
                fn(s_ax * sz, sz)

            @pl.when(s_ax == _NS - 1)
            def _tail():
                fn((_NS - 1) * sz, tail)

        def init_acc():
            spread(lambda st, ln: pltpu.sync_copy(
                zrows_hbm.at[pl.ds(st, ln)], acc.at[pl.ds(st, ln)]))

        def drain(out_a, out_b):
            def _go(st, ln):
                @pl.when(ci == 0)
                def _a():
                    pltpu.sync_copy(acc.at[pl.ds(st, ln)],
                                    out_a.at[pl.ds(st, ln)])

                @pl.when(ci == 1)
                def _b():
                    pltpu.sync_copy(acc.at[pl.ds(st, ln)],
                                    out_b.at[pl.ds(st, ln)])

            spread(_go)

        def didx_copy(j, p):
            return pltpu.make_async_copy(
                dst_hbm.at[pl.ds(base + j * c, c)], didx[p], sd[p])

        def sidx_copy(j, p):
            return pltpu.make_async_copy(
                src_hbm.at[pl.ds(base + j * c, c)], sidx[p], si_[p])

        def run_pass(mode, rows_hbm):
            def lrows_copy(j, p):
                return pltpu.make_async_copy(
                    rows_hbm.at[pl.ds(base + j * c, c)], rows[p], sg[p])

            def grows_copy(j, p):
                return pltpu.make_async_copy(
                    rows_hbm.at[sidx[p]], rows[p], sg[p])

            def ssrc(p):
                return rows[0] if mode == "cnt" else rows[p]

            def scatter_start(j, p):
                pltpu.async_copy(ssrc(p), acc.at[didx[p]], ss[p], add=True)

            def scatter_wait(j, p):
                pltpu.make_async_copy(ssrc(p), acc.at[didx[p]], ss[p]).wait()

            def stage1(k, p, guard_issue, guard_swait):
                def _swait():
                    scatter_wait(k - nb, p)

                def _start():
                    didx_copy(k, p).start()
                    if mode == "gather":
                        sidx_copy(k, p).start()
                    elif mode == "linear":
                        lrows_copy(k, p).start()

                if guard_swait:
                    pl.when(k >= nb)(_swait)
                else:
                    _swait()
                if guard_issue:
                    pl.when(k < nch)(_start)
                else:
                    _start()

            def stage2(k, p):
                if mode != "gather":
                    return
                sidx_copy(k, p).wait()
                grows_copy(k, p).start()

            def stage3(k, p):
                didx_copy(k, p).wait()
                if mode == "linear":
                    lrows_copy(k, p).wait()
                elif mode == "gather":
                    grows_copy(k, p).wait()
                scatter_start(k, p)

            stage1(0, 0, False, True)
            stage1(1, 1, False, True)
            stage2(0, 0)

            def step(jj, carry):
                kk = nb * jj
                for p in range(nb):
                    k = kk + p
                    stage1(k + 2, (p + 2) % nb,
                           guard_issue=(p == nb - 1), guard_swait=(p < 2))
                    stage2(k + 1, (p + 1) % nb)
                    stage3(k, p)
                return carry

            lax.fori_loop(0, (nch - 1) // nb, step, 0)
            stage3(nch - 1, (nch - 1) % nb)
            for back in range(3):
                j = nch - 1 - back
                scatter_wait(j, j % nb)

        pltpu.sync_copy(ones_hbm, rows[0])
        init_acc()
        plsc.subcore_barrier()
        run_pass("cnt", None)
        plsc.subcore_barrier()
        drain(cnt_a, cnt_b)
        plsc.subcore_barrier()
        run_pass("linear", hr_hbm)
        plsc.subcore_barrier()
        drain(hr_a, hr_b)
        plsc.subcore_barrier()
        run_pass("gather", he_hbm)
        plsc.subcore_barrier()
        drain(seg_a, seg_b)

    shp = jax.ShapeDtypeStruct((n, d), jnp.float32)
    return pl.kernel(
        body,
        mesh=mesh,
        out_type=(shp,) * 6,
        scratch_types=([pltpu.VMEM((c,), jnp.int32)] * (2 * nb)
                       + [pltpu.VMEM((c, d), jnp.float32)] * nb
                       + [pltpu.VMEM_SHARED((n, d), jnp.float32)]
                       + [pltpu.SemaphoreType.DMA] * (4 * nb)),
    )


def _mlp_body(h_ref, sa_ref, sb_ref, ha_ref, hb_ref, ca_ref, cb_ref,
              w_ref, b_ref, o_ref):
    d = h_ref.shape[1]
    csum = ca_ref[...] + cb_ref[...]
    hsum = ha_ref[...] + hb_ref[...]
    ssum = sa_ref[...] + sb_ref[...]
    cnt = csum[:, 0:1]
    inv = 1.0 / jnp.maximum(cnt, 1.0)
    he = (ssum - hsum) * inv
    hr = (hsum - csum) * inv
    w = w_ref[...]
    acc = jnp.dot(h_ref[...], w[:d],
                  preferred_element_type=jnp.float32,
                  precision=lax.Precision.HIGHEST)
    acc = acc + jnp.dot(he, w[d:2 * d],
                        preferred_element_type=jnp.float32,
                        precision=lax.Precision.HIGHEST)
    acc = acc + jnp.dot(hr, w[2 * d:],
                        preferred_element_type=jnp.float32,
                        precision=lax.Precision.HIGHEST)
    o_ref[...] = jnp.maximum(acc + b_ref[...], 0.0)


def _mlp(h, sega, segb, hra, hrb, cnta, cntb, w, b):
    n, d = h.shape
    bn = 1000
    grid = (n // bn,)

    def row_spec(width):
        return pl.BlockSpec((bn, width), lambda i: (i, 0))

    const_w = pl.BlockSpec((3 * d, d), lambda i: (0, 0))
    const_b = pl.BlockSpec((1, d), lambda i: (0, 0))
    return pl.pallas_call(
        _mlp_body,
        grid=grid,
        in_specs=[row_spec(d), row_spec(d), row_spec(d), row_spec(d),
                  row_spec(d), row_spec(d), row_spec(d), const_w, const_b],
        out_specs=row_spec(d),
        out_shape=jax.ShapeDtypeStruct((n, d), jnp.float32),
    )(h, sega, segb, hra, hrb, cnta, cntb,
      w, b.reshape(1, d))


def kernel(edge_index, topic_entity_one_hot, h_e, h_r, W1, b1, W2, b2):
    n, d = h_e.shape
    e = h_r.shape[0]
    c = 80
    src = edge_index[0]
    dst = edge_index[1]

    zrows = jnp.zeros((n, d), jnp.float32)

    ones = jnp.ones((c, d), jnp.float32)
    cnta, cntb, hra, hrb, s1a, s1b = _make_stage1_kernel(n, d, e, c)(
        src, dst, h_r, h_e, zrows, ones)
    gather = _make_scatter_kernel(n, d, e, c, "gather")
    h1 = _mlp(h_e, s1a, s1b, hra, hrb, cnta, cntb, W1, b1)
    s2a, s2b = gather(src, dst, h1, hra, hrb)
    h2 = _mlp(h1, s2a, s2b, hra, hrb, cnta, cntb, W2, b2)
    return (topic_entity_one_hot, h2)
